# baseline (speedup 1.0000x reference)
import functools

import jax
import jax.numpy as jnp
from jax import lax
from jax.experimental import pallas as pl
from jax.experimental.pallas import tpu as pltpu
from jax.experimental.pallas import tpu_sc as plsc

_H = 24
_W = 24
_D2 = 384
_D = 768
_P = _H * _W
_L = 16

_BB = 16
_SPLIT = 16


def _pos_body(row_hbm, col_hbm, pos_hbm, re_v, pos_v, sem):
    wid = lax.axis_index("s") * 2 + lax.axis_index("c")

    @pl.when(wid < _H)
    def _():
        r = wid
        ccol = pltpu.async_copy(
            col_hbm.at[pl.ds(0, _W)], pos_v.at[:, pl.ds(_D2, _D2)], sem
        )
        pltpu.sync_copy(row_hbm.at[r], re_v)
        for i in range(_D2 // _L):
            v = re_v[pl.ds(i * _L, _L)]
            for c in range(_W):
                pos_v[c, pl.ds(i * _L, _L)] = v
        ccol.wait()
        pltpu.sync_copy(pos_v, pos_hbm.at[r])


@functools.partial(
    pl.kernel,
    out_type=jax.ShapeDtypeStruct((_H, _W, _D), jnp.float32),
    mesh=plsc.VectorSubcoreMesh(core_axis_name="c", subcore_axis_name="s"),
    scratch_types=[
        pltpu.VMEM((_D2,), jnp.float32),
        pltpu.VMEM((_W, _D), jnp.float32),
        pltpu.SemaphoreType.DMA,
    ],
)
def _build_pos(row_hbm, col_hbm, pos_hbm, re_v, pos_v, sem):
    _pos_body(row_hbm, col_hbm, pos_hbm, re_v, pos_v, sem)


def _add_direct(row_ref, col_ref, x_ref, o_ref):
    xb = x_ref[...]
    o_ref[:, :, :, : _D2] = xb[:, :, :, : _D2] + row_ref[...][None, :, None, :]
    o_ref[:, :, :, _D2:] = xb[:, :, :, _D2:] + col_ref[...][None, None, :, :]


def _add_pos(prev_ref, pos_ref, x_ref, o_ref):
    del prev_ref
    o_ref[...] = x_ref[...] + pos_ref[...][None]


def kernel(x, h, w, row_embed, col_embed):
    B, P, D = x.shape
    x4 = x.reshape(B, _H, _W, D)
    n_a = _SPLIT // _BB
    n_b = (B - _SPLIT) // _BB

    pos = _build_pos(row_embed, col_embed)

    out_a = pl.pallas_call(
        _add_direct,
        grid=(n_a,),
        in_specs=[
            pl.BlockSpec((_H, _D2), lambda b: (0, 0)),
            pl.BlockSpec((_W, _D2), lambda b: (0, 0)),
            pl.BlockSpec((_BB, _H, _W, D), lambda b: (b, 0, 0, 0)),
        ],
        out_specs=pl.BlockSpec((_BB, _H, _W, D), lambda b: (b, 0, 0, 0)),
        out_shape=jax.ShapeDtypeStruct((B, _H, _W, D), jnp.float32),
    )(row_embed, col_embed, x4)

    out = pl.pallas_call(
        _add_pos,
        grid=(n_b,),
        in_specs=[
            pl.BlockSpec(memory_space=pl.ANY),
            pl.BlockSpec((_H, _W, D), lambda b: (0, 0, 0)),
            pl.BlockSpec((_BB, _H, _W, D), lambda b: (b + n_a, 0, 0, 0)),
        ],
        out_specs=pl.BlockSpec((_BB, _H, _W, D), lambda b: (b + n_a, 0, 0, 0)),
        out_shape=jax.ShapeDtypeStruct((B, _H, _W, D), jnp.float32),
        input_output_aliases={0: 0},
    )(out_a, pos, x4)
    return out.reshape(B, P, D)


# speedup vs baseline: 1.0791x; 1.0791x over previous
import functools

import jax
import jax.numpy as jnp
from jax import lax
from jax.experimental import pallas as pl
from jax.experimental.pallas import tpu as pltpu
from jax.experimental.pallas import tpu_sc as plsc

_H = 24
_W = 24
_D2 = 384
_D = 768
_P = _H * _W
_L = 16

_BB = 8
_SPLIT = 16


def _pos_body(row_hbm, col_hbm, pos_hbm, re_v, pos_v, sem):
    wid = lax.axis_index("s") * 2 + lax.axis_index("c")

    @pl.when(wid < _H)
    def _():
        r = wid
        ccol = pltpu.async_copy(
            col_hbm.at[pl.ds(0, _W)], pos_v.at[:, pl.ds(_D2, _D2)], sem
        )
        pltpu.sync_copy(row_hbm.at[r], re_v)
        for i in range(_D2 // _L):
            v = re_v[pl.ds(i * _L, _L)]
            for c in range(_W):
                pos_v[c, pl.ds(i * _L, _L)] = v
        ccol.wait()
        pltpu.sync_copy(pos_v, pos_hbm.at[r])


@functools.partial(
    pl.kernel,
    out_type=jax.ShapeDtypeStruct((_H, _W, _D), jnp.float32),
    mesh=plsc.VectorSubcoreMesh(core_axis_name="c", subcore_axis_name="s"),
    scratch_types=[
        pltpu.VMEM((_D2,), jnp.float32),
        pltpu.VMEM((_W, _D), jnp.float32),
        pltpu.SemaphoreType.DMA,
    ],
)
def _build_pos(row_hbm, col_hbm, pos_hbm, re_v, pos_v, sem):
    _pos_body(row_hbm, col_hbm, pos_hbm, re_v, pos_v, sem)


def _add_direct(row_ref, col_ref, x_ref, o_ref):
    xb = x_ref[...]
    o_ref[:, :, :, : _D2] = xb[:, :, :, : _D2] + row_ref[...][None, :, None, :]
    o_ref[:, :, :, _D2:] = xb[:, :, :, _D2:] + col_ref[...][None, None, :, :]


def _add_pos(prev_ref, pos_ref, x_ref, o_ref):
    del prev_ref
    o_ref[...] = x_ref[...] + pos_ref[...][None]


def kernel(x, h, w, row_embed, col_embed):
    B, P, D = x.shape
    x4 = x.reshape(B, _H, _W, D)
    n_a = _SPLIT // _BB
    n_b = (B - _SPLIT) // _BB

    pos = _build_pos(row_embed, col_embed)

    out_a = pl.pallas_call(
        _add_direct,
        grid=(n_a,),
        in_specs=[
            pl.BlockSpec((_H, _D2), lambda b: (0, 0)),
            pl.BlockSpec((_W, _D2), lambda b: (0, 0)),
            pl.BlockSpec((_BB, _H, _W, D), lambda b: (b, 0, 0, 0)),
        ],
        out_specs=pl.BlockSpec((_BB, _H, _W, D), lambda b: (b, 0, 0, 0)),
        out_shape=jax.ShapeDtypeStruct((B, _H, _W, D), jnp.float32),
    )(row_embed, col_embed, x4)

    out = pl.pallas_call(
        _add_pos,
        grid=(n_b,),
        in_specs=[
            pl.BlockSpec(memory_space=pl.ANY),
            pl.BlockSpec((_H, _W, D), lambda b: (0, 0, 0)),
            pl.BlockSpec((_BB, _H, _W, D), lambda b: (b + n_a, 0, 0, 0)),
        ],
        out_specs=pl.BlockSpec((_BB, _H, _W, D), lambda b: (b + n_a, 0, 0, 0)),
        out_shape=jax.ShapeDtypeStruct((B, _H, _W, D), jnp.float32),
        input_output_aliases={0: 0},
    )(out_a, pos, x4)
    return out.reshape(B, P, D)


# speedup vs baseline: 1.6928x; 1.5687x over previous
import functools

import jax
import jax.numpy as jnp
from jax import lax
from jax.experimental import pallas as pl
from jax.experimental.pallas import tpu as pltpu
from jax.experimental.pallas import tpu_sc as plsc

_H = 24
_W = 24
_D2 = 384
_D = 768
_P = _H * _W
_L = 16

_BB = 8
_SPLIT = 16


def _pos_body(row_hbm, col_hbm, pos_hbm, re_v, pos_v, sem):
    wid = lax.axis_index("s") * 2 + lax.axis_index("c")

    @pl.when(wid < _H)
    def _():
        r = wid
        ccol = pltpu.async_copy(
            col_hbm.at[pl.ds(0, _W)], pos_v.at[:, pl.ds(_D2, _D2)], sem
        )
        pltpu.sync_copy(row_hbm.at[r], re_v)
        for i in range(_D2 // _L):
            v = re_v[pl.ds(i * _L, _L)]
            for c in range(_W):
                pos_v[c, pl.ds(i * _L, _L)] = v
        ccol.wait()
        pltpu.sync_copy(pos_v, pos_hbm.at[r])


@functools.partial(
    pl.kernel,
    out_type=jax.ShapeDtypeStruct((_H, _W, _D), jnp.float32),
    mesh=plsc.VectorSubcoreMesh(core_axis_name="c", subcore_axis_name="s"),
    scratch_types=[
        pltpu.VMEM((_D2,), jnp.float32),
        pltpu.VMEM((_W, _D), jnp.float32),
        pltpu.SemaphoreType.DMA,
    ],
)
def _build_pos(row_hbm, col_hbm, pos_hbm, re_v, pos_v, sem):
    _pos_body(row_hbm, col_hbm, pos_hbm, re_v, pos_v, sem)


def _add_direct(row_ref, col_ref, x_ref, o_ref):
    xb = x_ref[...]
    o_ref[:, :, :, : _D2] = xb[:, :, :, : _D2] + row_ref[...][None, :, None, :]
    o_ref[:, :, :, _D2:] = xb[:, :, :, _D2:] + col_ref[...][None, None, :, :]


def _add_pos(prev_ref, pos_ref, x_ref, o_ref):
    del prev_ref
    o_ref[...] = x_ref[...] + pos_ref[...][None]


def kernel(x, h, w, row_embed, col_embed):
    B, P, D = x.shape
    x4 = x.reshape(B, _H, _W, D)
    n_a = _SPLIT // _BB
    n_b = (B - _SPLIT) // _BB


    out_a = pl.pallas_call(
        _add_direct,
        grid=(n_a,),
        in_specs=[
            pl.BlockSpec((_H, _D2), lambda b: (0, 0)),
            pl.BlockSpec((_W, _D2), lambda b: (0, 0)),
            pl.BlockSpec((_BB, _H, _W, D), lambda b: (b, 0, 0, 0)),
        ],
        out_specs=pl.BlockSpec((_BB, _H, _W, D), lambda b: (b, 0, 0, 0)),
        out_shape=jax.ShapeDtypeStruct((B, _H, _W, D), jnp.float32),
    )(row_embed, col_embed, x4)

    out = pl.pallas_call(
        lambda p_ref, r_ref, c_ref, x_ref, o_ref: _add_direct(r_ref, c_ref, x_ref, o_ref),
        grid=(n_b,),
        in_specs=[
            pl.BlockSpec(memory_space=pl.ANY),
            pl.BlockSpec((_H, _D2), lambda b: (0, 0)),
            pl.BlockSpec((_W, _D2), lambda b: (0, 0)),
            pl.BlockSpec((_BB, _H, _W, D), lambda b: (b + n_a, 0, 0, 0)),
        ],
        out_specs=pl.BlockSpec((_BB, _H, _W, D), lambda b: (b + n_a, 0, 0, 0)),
        out_shape=jax.ShapeDtypeStruct((B, _H, _W, D), jnp.float32),
        input_output_aliases={0: 0},
    )(out_a, row_embed, col_embed, x4)
    return out.reshape(B, P, D)
